# baseline (speedup 1.0000x reference)
import jax
import jax.numpy as jnp
import numpy as np
from jax import lax
from jax.experimental import pallas as pl
from jax.experimental.pallas import tpu as pltpu
from jax.experimental.pallas import tpu_sc as plsc

N_NODES = 10000
N_EDGES = 320000
H = 64
CUTOFF = 5.0

NW = 32
LANE = 128
KJ = 8
CH = KJ * LANE
NITER_E = 10
EP = NW * NITER_E * CH
NP = 10240
SUB = 512

_PI = float(np.pi)
_BES_SCALE = float(np.sqrt(2.0 / CUTOFF))


def _sigm(x):
    return 0.5 * (1.0 + jnp.tanh(0.5 * x))


def _silu(x):
    return x * _sigm(x)


def _sc_gather_elem(tables, idx_rows, niter):
    nt = len(tables)
    nrows = idx_rows.shape[0]
    b_total = nrows * LANE
    mesh = plsc.VectorSubcoreMesh(core_axis_name="c", subcore_axis_name="s")

    def body(*refs):
        t_refs = refs[:nt]
        idx_hbm = refs[nt]
        o_refs = refs[nt + 1:nt + 1 + nt]
        idx_v = refs[nt + 1 + nt]
        row_bufs = refs[nt + 2 + nt:nt + 2 + 2 * nt]
        sem, wsem = refs[nt + 2 + 2 * nt:]
        c = lax.axis_index("c")
        s = lax.axis_index("s")
        wid = s * 2 + c

        def step(it, carry):
            rowbase = (wid * niter + it) * KJ
            base = (wid * niter + it) * CH
            pltpu.sync_copy(idx_hbm.at[pl.ds(rowbase, KJ)], idx_v)
            outs = []
            for t in range(nt):
                cps = [
                    pltpu.async_copy(
                        t_refs[t].at[idx_v.at[j]],
                        row_bufs[t].at[pl.ds(j * LANE, LANE)],
                        sem,
                    )
                    for j in range(KJ)
                ]
                for cp in cps:
                    cp.wait()
                outs.append(pltpu.async_copy(
                    row_bufs[t], o_refs[t].at[pl.ds(base, CH)], wsem))
            for o in outs:
                o.wait()
            return carry

        lax.fori_loop(0, niter, step, 0)

    return pl.kernel(
        body,
        out_type=[jax.ShapeDtypeStruct((b_total,), jnp.float32)] * nt,
        mesh=mesh,
        compiler_params=pltpu.CompilerParams(use_tc_tiling_on_sc=False),
        scratch_types=[pltpu.VMEM((KJ, LANE), jnp.int32)]
        + [pltpu.VMEM((CH,), jnp.float32)] * nt
        + [pltpu.SemaphoreType.DMA, pltpu.SemaphoreType.DMA],
    )(*tables, idx_rows)


def _sc_gather128(table, idx_rows):
    niter = NITER_E
    nrows = idx_rows.shape[0]
    b_total = nrows * LANE
    mesh = plsc.VectorSubcoreMesh(core_axis_name="c", subcore_axis_name="s")

    SUBG = 256

    def body(table_hbm, idx_hbm, out_hbm, idx_v, rows_a, rows_b, sem, wsa, wsb):
        c = lax.axis_index("c")
        s = lax.axis_index("s")
        wid = s * 2 + c
        bufs = (rows_a, rows_b)
        wsems = (wsa, wsb)

        def drain(b):
            pltpu.make_async_copy(
                out_hbm.at[pl.ds(0, SUBG)], bufs[b], wsems[b]).wait()

        def step(it, carry):
            rowbase = (wid * niter + it) * KJ
            base = (wid * niter + it) * CH
            pltpu.sync_copy(idx_hbm.at[pl.ds(rowbase, KJ)], idx_v)
            for h in range(CH // SUBG):
                b = h % 2
                if h >= 2:
                    drain(b)
                else:
                    @pl.when(it > 0)
                    def _():
                        drain(b)
                cps = [
                    pltpu.async_copy(
                        table_hbm.at[idx_v.at[h * (SUBG // LANE) + j]],
                        bufs[b].at[pl.ds(j * LANE, LANE)],
                        sem,
                    )
                    for j in range(SUBG // LANE)
                ]
                for cp in cps:
                    cp.wait()
                pltpu.async_copy(
                    bufs[b], out_hbm.at[pl.ds(base + h * SUBG, SUBG)],
                    wsems[b])
            return carry

        lax.fori_loop(0, niter, step, 0)
        drain(0)
        drain(1)

    return pl.kernel(
        body,
        out_type=jax.ShapeDtypeStruct((b_total, 2 * H), jnp.float32),
        mesh=mesh,
        scratch_types=[
            pltpu.VMEM((KJ, LANE), jnp.int32),
            pltpu.VMEM((SUBG, 2 * H), jnp.float32),
            pltpu.VMEM((SUBG, 2 * H), jnp.float32),
            pltpu.SemaphoreType.DMA,
            pltpu.SemaphoreType.DMA,
            pltpu.SemaphoreType.DMA,
        ],
    )(table, idx_rows)


def _sc_scatter_add128(upd, idx_rows, zeros_np):
    niter = NITER_E
    mesh = plsc.VectorSubcoreMesh(core_axis_name="c", subcore_axis_name="s")
    rps = NP // 16

    def body(upd_hbm, idx_hbm, z_hbm, out_hbm, idx_v, ubuf_a, ubuf_b, shared,
             sem):
        c = lax.axis_index("c")
        s = lax.axis_index("s")
        wid = s * 2 + c
        pltpu.sync_copy(z_hbm.at[pl.ds(s * rps, rps)], shared.at[pl.ds(s * rps, rps)])
        plsc.subcore_barrier()
        bufs = (ubuf_a, ubuf_b)

        def step(it, carry):
            rowbase = (wid * niter + it) * KJ
            base = (wid * niter + it) * CH
            pltpu.sync_copy(idx_hbm.at[pl.ds(rowbase, KJ)], idx_v)
            cur = pltpu.async_copy(upd_hbm.at[pl.ds(base, LANE)], bufs[0], sem)
            for j in range(KJ):
                b = j % 2
                nxt = None
                if j + 1 < KJ:
                    nxt = pltpu.async_copy(
                        upd_hbm.at[pl.ds(base + (j + 1) * LANE, LANE)],
                        bufs[1 - b], sem)
                cur.wait()
                pltpu.sync_copy(bufs[b], shared.at[idx_v.at[j]], add=True)
                cur = nxt
            return carry

        lax.fori_loop(0, niter, step, 0)
        plsc.subcore_barrier()
        pltpu.sync_copy(
            shared.at[pl.ds(s * rps, rps)],
            out_hbm.at[pl.ds(c * NP + s * rps, rps)],
        )

    return pl.kernel(
        body,
        out_type=jax.ShapeDtypeStruct((2 * NP, 2 * H), jnp.float32),
        mesh=mesh,
        scratch_types=[
            pltpu.VMEM((KJ, LANE), jnp.int32),
            pltpu.VMEM((LANE, 2 * H), jnp.float32),
            pltpu.VMEM((LANE, 2 * H), jnp.float32),
            pltpu.VMEM_SHARED((NP, 2 * H), jnp.float32),
            pltpu.SemaphoreType.DMA,
        ],
    )(upd, idx_rows, zeros_np)


BE = 2048
BR = BE // 128


def _full(shape):
    return pl.BlockSpec(shape, lambda i: tuple(0 for _ in shape))


def _expand64(dense_ref):
    return jnp.concatenate(
        [jnp.broadcast_to(dense_ref[r:r + 1, :].T, (LANE, H)) for r in range(BR)],
        axis=0)


def _edge_featurize(comps, r1w0, r1b0, r2w0, r2b0, r1w1, r1b1, r2w1, r2b1,
                    or1w, or1b, or2w, or2b, epw, epb, awc):
    grid = (EP // BE,)

    def body(sx_ref, sy_ref, sz_ref, dx_ref, dy_ref, dz_ref,
             w10_ref, b10_ref, w20_ref, b20_ref,
             w11_ref, b11_ref, w21_ref, b21_ref,
             ow1_ref, ob1_ref, ow2_ref, ob2_ref,
             wm_ref, bm_ref, awc_ref,
             x1_ref, x2_ref, aw1_ref, aw2_ref):
        vx = sx_ref[...] - dx_ref[...]
        vy = sy_ref[...] - dy_ref[...]
        vz = sz_ref[...] - dz_ref[...]
        d2 = vx * vx + vy * vy + vz * vz + 1e-12
        inv = lax.rsqrt(d2)
        dd = d2 * inv
        x = vx * inv
        y = vy * inv
        z = vz * inv
        xc = dd * (1.0 / CUTOFF)
        xc2 = xc * xc
        xc6 = (xc2 * xc) * (xc2 * xc)
        env = 1.0 - 28.0 * xc6 + 48.0 * xc6 * xc - 21.0 * xc6 * xc2
        env = env * (dd < CUTOFF).astype(jnp.float32)
        a = dd * (_PI / CUTOFF)
        s1 = jnp.sin(a)
        c2 = 2.0 * jnp.cos(a)
        scale = (_BES_SCALE * inv) * env
        rbfs = [scale * s1]
        sprev, scur = jnp.zeros_like(s1), s1
        for _ in range(15):
            sprev, scur = scur, c2 * scur - sprev
            rbfs.append(scale * scur)
        awc = awc_ref[...]
        z2t = 3.0 * z * z - 1.0
        for b, ref in ((0, aw1_ref), (1, aw2_ref)):
            lin = (awc[0, b] + awc[1, b] * y + awc[2, b] * z + awc[3, b] * x
                   + awc[4, b] * x * y + awc[5, b] * y * z + awc[6, b] * z2t
                   + awc[7, b])
            ref[...] = _sigm(lin)
        pieces = []
        for r in range(BR):
            rt = jnp.concatenate([rb[r:r + 1, :] for rb in rbfs], axis=0)
            pieces.append(rt.T)
        R = jnp.concatenate(pieces, axis=0)
        m0 = jnp.dot(R, wm_ref[...]) + bm_ref[...]
        rf1 = jnp.dot(_silu(jnp.dot(R, w10_ref[...]) + b10_ref[...]),
                      w20_ref[...]) + b20_ref[...]
        rf2 = jnp.dot(_silu(jnp.dot(R, w11_ref[...]) + b11_ref[...]),
                      w21_ref[...]) + b21_ref[...]
        wout = jnp.dot(_silu(jnp.dot(R, ow1_ref[...]) + ob1_ref[...]),
                       ow2_ref[...]) + ob2_ref[...]
        x1_ref[...] = jnp.concatenate([m0 * rf1, wout], axis=1)
        x2_ref[...] = jnp.concatenate([rf2, m0], axis=1)

    dspec = pl.BlockSpec((BR, LANE), lambda i: (i, 0))
    return pl.pallas_call(
        body,
        grid=grid,
        in_specs=[dspec] * 6 + [
            _full((16, H)), _full((1, H)), _full((H, H)), _full((1, H)),
            _full((16, H)), _full((1, H)), _full((H, H)), _full((1, H)),
            _full((16, H)), _full((1, H)), _full((H, H)), _full((1, H)),
            _full((16, H)), _full((1, H)), _full((8, 2)),
        ],
        out_specs=[
            pl.BlockSpec((BE, 2 * H), lambda i: (i, 0)),
            pl.BlockSpec((BE, 2 * H), lambda i: (i, 0)),
            dspec, dspec,
        ],
        out_shape=[
            jax.ShapeDtypeStruct((EP, 2 * H), jnp.float32),
            jax.ShapeDtypeStruct((EP, 2 * H), jnp.float32),
            jax.ShapeDtypeStruct((EP // LANE, LANE), jnp.float32),
            jax.ShapeDtypeStruct((EP // LANE, LANE), jnp.float32),
        ],
    )(*comps, r1w0, r1b0, r2w0, r2b0, r1w1, r1b1, r2w1, r2b1,
      or1w, or1b, or2w, or2b, epw, epb, awc)


def _block1_triplet(G1, Gaw1d, X2, ow, obs):
    grid = (EP // BE,)

    l01 = jnp.concatenate([ow[0], ow[1]], axis=1)
    b01 = jnp.concatenate([obs[0:1, :], obs[1:2, :]], axis=1)

    def body(gmr_ref, gaw_ref, x2_ref,
             l01_ref, b01_ref, l2_ref, b2_ref, x5_ref):
        gid = pl.program_id(0)
        rows = gid * BE + lax.broadcasted_iota(jnp.int32, (BE, 1), 0)
        valid = (rows < N_EDGES).astype(jnp.float32)
        aw64 = _expand64(gaw_ref)
        g1 = gmr_ref[:, 0:H] * aw64 * valid
        a = _silu(jnp.dot(g1, l01_ref[...]) + b01_ref[...])
        mn = (a[:, 0:H] + a[:, H:2 * H]
              + _silu(jnp.dot(g1, l2_ref[...]) + b2_ref[...]))
        x2 = x2_ref[...]
        mr2 = (x2[:, H:2 * H] + mn) * x2[:, 0:H]
        x5_ref[...] = jnp.concatenate([g1, mr2], axis=1)

    return pl.pallas_call(
        body,
        grid=grid,
        in_specs=[
            pl.BlockSpec((BE, 2 * H), lambda i: (i, 0)),
            pl.BlockSpec((BR, LANE), lambda i: (i, 0)),
            pl.BlockSpec((BE, 2 * H), lambda i: (i, 0)),
            _full((H, 2 * H)), _full((1, 2 * H)), _full((H, H)), _full((1, H)),
        ],
        out_specs=pl.BlockSpec((BE, 2 * H), lambda i: (i, 0)),
        out_shape=jax.ShapeDtypeStruct((EP, 2 * H), jnp.float32),
    )(G1, Gaw1d, X2, l01, b01, ow[2], obs[2:3, :])


def _block2_messages(G2, Gaw2d):
    grid = (EP // BE,)

    def body(gmr_ref, gaw_ref, x7_ref):
        gid = pl.program_id(0)
        rows = gid * BE + lax.broadcasted_iota(jnp.int32, (BE, 1), 0)
        valid = (rows < N_EDGES).astype(jnp.float32)
        g2 = gmr_ref[:, H:2 * H] * _expand64(gaw_ref) * valid
        x7_ref[...] = jnp.concatenate(
            [g2, jnp.zeros((BE, H), jnp.float32)], axis=1)

    return pl.pallas_call(
        body,
        grid=grid,
        in_specs=[
            pl.BlockSpec((BE, 2 * H), lambda i: (i, 0)),
            pl.BlockSpec((BR, LANE), lambda i: (i, 0)),
        ],
        out_specs=pl.BlockSpec((BE, 2 * H), lambda i: (i, 0)),
        out_shape=jax.ShapeDtypeStruct((EP, 2 * H), jnp.float32),
    )(G2, Gaw2d)


def _out_edges(GH, X1):
    grid = (EP // BE,)

    def body(gh_ref, wo_ref, x9_ref):
        gid = pl.program_id(0)
        rows = gid * BE + lax.broadcasted_iota(jnp.int32, (BE, 1), 0)
        valid = (rows < N_EDGES).astype(jnp.float32)
        he = gh_ref[:, 0:H] * wo_ref[:, H:2 * H] * valid
        x9_ref[...] = jnp.concatenate(
            [he, jnp.zeros((BE, H), jnp.float32)], axis=1)

    return pl.pallas_call(
        body,
        grid=grid,
        in_specs=[
            pl.BlockSpec((BE, 2 * H), lambda i: (i, 0)),
            pl.BlockSpec((BE, 2 * H), lambda i: (i, 0)),
        ],
        out_specs=pl.BlockSpec((BE, 2 * H), lambda i: (i, 0)),
        out_shape=jax.ShapeDtypeStruct((EP, 2 * H), jnp.float32),
    )(GH, X1)


def _node_update(h, aggp, wh, wa, b1, w2, b2, duplicate_out):
    def body(h_ref, a0_ref, a1_ref, wh_ref, wa_ref, b1_ref, w2_ref, b2_ref,
             o_ref):
        agg = a0_ref[0:N_NODES, 0:H] + a1_ref[0:N_NODES, 0:H]
        h = h_ref[...]
        pre = _silu(jnp.dot(h, wh_ref[...]) + jnp.dot(agg, wa_ref[...])
                    + b1_ref[...])
        hn = h + jnp.dot(pre, w2_ref[...]) + b2_ref[...]
        if duplicate_out:
            o_ref[...] = jnp.concatenate([hn, hn], axis=1)
        else:
            o_ref[...] = hn

    return pl.pallas_call(
        body,
        grid=(1,),
        in_specs=[
            _full((N_NODES, H)),
            pl.BlockSpec((NP, 2 * H), lambda i: (0, 0)),
            pl.BlockSpec((NP, 2 * H), lambda i: (1, 0)),
            _full((H, H)), _full((H, H)), _full((1, H)), _full((H, H)),
            _full((1, H)),
        ],
        out_specs=_full((N_NODES, 2 * H if duplicate_out else H)),
        out_shape=jax.ShapeDtypeStruct(
            (N_NODES, 2 * H if duplicate_out else H), jnp.float32),
    )(h, aggp, aggp, wh, wa, b1, w2, b2)


def _out_mlp(aggp, d0w, d0b, d1w, d1b, d2w, d2b, fw, fb):
    def body(a0_ref, a1_ref, d0w_ref, d0b_ref, d1w_ref, d1b_ref,
             d2w_ref, d2b_ref, fw_ref, fb_ref, o_ref):
        xx = a0_ref[0:N_NODES, 0:H] + a1_ref[0:N_NODES, 0:H]
        xx = _silu(jnp.dot(xx, d0w_ref[...]) + d0b_ref[...])
        xx = _silu(jnp.dot(xx, d1w_ref[...]) + d1b_ref[...])
        xx = _silu(jnp.dot(xx, d2w_ref[...]) + d2b_ref[...])
        o_ref[...] = jnp.dot(xx, fw_ref[...]) + fb_ref[...]

    return pl.pallas_call(
        body,
        grid=(1,),
        in_specs=[
            pl.BlockSpec((NP, 2 * H), lambda i: (0, 0)),
            pl.BlockSpec((NP, 2 * H), lambda i: (1, 0)),
            _full((H, H)), _full((1, H)), _full((H, H)), _full((1, H)),
            _full((H, H)), _full((1, H)), _full((H, 1)), _full((1, 1)),
        ],
        out_specs=_full((N_NODES, 1)),
        out_shape=jax.ShapeDtypeStruct((N_NODES, 1), jnp.float32),
    )(aggp, aggp, d0w, d0b, d1w, d1b, d2w, d2b, fw, fb)


def kernel(atomic_numbers, positions, edge_index, triplets, params):
    src = edge_index[0]
    dst = edge_index[1]
    ji = triplets[:, 0]
    kj = triplets[:, 1]

    padE = EP - N_EDGES
    arE = jnp.arange(padE, dtype=jnp.int32)
    src_p = jnp.concatenate([src, arE % N_NODES])
    dst_p = jnp.concatenate([dst, arE % N_NODES])
    ji_p = jnp.concatenate([ji, arE % N_EDGES])
    kj_p = jnp.concatenate([kj, arE % N_EDGES])
    ji_r = ji_p.reshape(EP // LANE, LANE)
    kj_r = kj_p.reshape(EP // LANE, LANE)
    src_r = src_p.reshape(EP // LANE, LANE)
    dst_r = dst_p.reshape(EP // LANE, LANE)

    posx = positions[:, 0]
    posy = positions[:, 1]
    posz = positions[:, 2]
    zeros_np = jnp.zeros((NP, 2 * H), jnp.float32)
    h0 = params["atom_emb"][atomic_numbers - 1]

    b0, b1 = params["blocks"]
    ep_w = params["edge_proj"]["w"][:, 2 * H:3 * H]
    ep_b = params["edge_proj"]["b"][2 * H:3 * H][None, :]

    def lin(p):
        return p["w"], p["b"][None, :]

    r1w0, r1b0 = lin(b0["radial1"])
    r2w0, r2b0 = lin(b0["radial2"])
    r1w1, r1b1 = lin(b1["radial1"])
    r2w1, r2b1 = lin(b1["radial2"])
    awc = jnp.stack(
        [jnp.concatenate([jnp.mean(b["sph"]["w"], axis=1),
                          jnp.mean(b["sph"]["b"])[None]])
         for b in (b0, b1)], axis=1)
    ob = params["out_block"]
    or1w, or1b = lin(ob["radial1"])
    or2w, or2b = lin(ob["radial2"])
    out_ws0 = [l["w"] for l in b0["out"]]
    out_bs0 = jnp.stack([l["b"] for l in b0["out"]])

    comps1d = (list(_sc_gather_elem([posx, posy, posz], src_r, NITER_E))
               + list(_sc_gather_elem([posx, posy, posz], dst_r, NITER_E)))
    comps = [c.reshape(EP // LANE, LANE) for c in comps1d]
    X1, X2, aw1d, aw2d = _edge_featurize(
        comps, r1w0, r1b0, r2w0, r2b0, r1w1, r1b1, r2w1, r2b1,
        or1w, or1b, or2w, or2b, ep_w, ep_b, awc)
    G1 = _sc_gather128(X1, kj_r)
    Gaw1, Gaw2 = _sc_gather_elem(
        [aw1d.reshape(EP), aw2d.reshape(EP)], ji_r, NITER_E)
    Gaw1d = Gaw1.reshape(EP // LANE, LANE)
    Gaw2d = Gaw2.reshape(EP // LANE, LANE)
    X5 = _block1_triplet(G1, Gaw1d, X2, out_ws0, out_bs0)
    agg1p = _sc_scatter_add128(X5, dst_r, zeros_np)
    u0w = b0["upd1"]["w"]
    h1 = _node_update(h0, agg1p, u0w[:H], u0w[H:], b0["upd1"]["b"][None, :],
                      b0["upd2"]["w"], b0["upd2"]["b"][None, :], False)
    G2 = _sc_gather128(X5, kj_r)
    X7 = _block2_messages(G2, Gaw2d)
    agg2p = _sc_scatter_add128(X7, dst_r, zeros_np)
    u1w = b1["upd1"]["w"]
    h2x = _node_update(h1, agg2p, u1w[:H], u1w[H:], b1["upd1"]["b"][None, :],
                       b1["upd2"]["w"], b1["upd2"]["b"][None, :], True)
    GH = _sc_gather128(h2x, src_r)
    X9 = _out_edges(GH, X1)
    agg3p = _sc_scatter_add128(X9, dst_r, zeros_np)
    d0, d1, d2 = ob["dense"]
    return _out_mlp(agg3p, d0["w"], d0["b"][None, :], d1["w"], d1["b"][None, :],
                    d2["w"], d2["b"][None, :], ob["final"]["w"],
                    ob["final"]["b"][None, :])


# speedup vs baseline: 1.0795x; 1.0795x over previous
import jax
import jax.numpy as jnp
import numpy as np
from jax import lax
from jax.experimental import pallas as pl
from jax.experimental.pallas import tpu as pltpu
from jax.experimental.pallas import tpu_sc as plsc

N_NODES = 10000
N_EDGES = 320000
H = 64
CUTOFF = 5.0

NW = 32
LANE = 128
KJ = 8
CH = KJ * LANE
NITER_E = 10
EP = NW * NITER_E * CH
NP = 10240
SUB = 512

_PI = float(np.pi)
_BES_SCALE = float(np.sqrt(2.0 / CUTOFF))


def _sigm(x):
    return 0.5 * (1.0 + jnp.tanh(0.5 * x))


def _silu(x):
    return x * _sigm(x)


def _sc_gather_elem(tables, idx_rows, niter):
    nt = len(tables)
    nrows = idx_rows.shape[0]
    b_total = nrows * LANE
    mesh = plsc.VectorSubcoreMesh(core_axis_name="c", subcore_axis_name="s")

    def body(*refs):
        t_refs = refs[:nt]
        idx_hbm = refs[nt]
        o_refs = refs[nt + 1:nt + 1 + nt]
        idx_v = refs[nt + 1 + nt]
        row_bufs = refs[nt + 2 + nt:nt + 2 + 2 * nt]
        sem, wsem = refs[nt + 2 + 2 * nt:]
        c = lax.axis_index("c")
        s = lax.axis_index("s")
        wid = s * 2 + c

        def step(it, carry):
            rowbase = (wid * niter + it) * KJ
            base = (wid * niter + it) * CH
            pltpu.sync_copy(idx_hbm.at[pl.ds(rowbase, KJ)], idx_v)
            outs = []
            for t in range(nt):
                cps = [
                    pltpu.async_copy(
                        t_refs[t].at[idx_v.at[j]],
                        row_bufs[t].at[pl.ds(j * LANE, LANE)],
                        sem,
                    )
                    for j in range(KJ)
                ]
                for cp in cps:
                    cp.wait()
                outs.append(pltpu.async_copy(
                    row_bufs[t], o_refs[t].at[pl.ds(base, CH)], wsem))
            for o in outs:
                o.wait()
            return carry

        lax.fori_loop(0, niter, step, 0)

    return pl.kernel(
        body,
        out_type=[jax.ShapeDtypeStruct((b_total,), jnp.float32)] * nt,
        mesh=mesh,
        compiler_params=pltpu.CompilerParams(use_tc_tiling_on_sc=False),
        scratch_types=[pltpu.VMEM((KJ, LANE), jnp.int32)]
        + [pltpu.VMEM((CH,), jnp.float32)] * nt
        + [pltpu.SemaphoreType.DMA, pltpu.SemaphoreType.DMA],
    )(*tables, idx_rows)


def _sc_gather128(table, idx_rows):
    niter = NITER_E
    nrows = idx_rows.shape[0]
    b_total = nrows * LANE
    mesh = plsc.VectorSubcoreMesh(core_axis_name="c", subcore_axis_name="s")

    SUBG = 256

    def body(table_hbm, idx_hbm, out_hbm, idx_v, rows_a, rows_b, sem, wsa, wsb):
        c = lax.axis_index("c")
        s = lax.axis_index("s")
        wid = s * 2 + c
        bufs = (rows_a, rows_b)
        wsems = (wsa, wsb)

        def drain(b):
            pltpu.make_async_copy(
                out_hbm.at[pl.ds(0, SUBG)], bufs[b], wsems[b]).wait()

        def step(it, carry):
            rowbase = (wid * niter + it) * KJ
            base = (wid * niter + it) * CH
            pltpu.sync_copy(idx_hbm.at[pl.ds(rowbase, KJ)], idx_v)
            for h in range(CH // SUBG):
                b = h % 2
                if h >= 2:
                    drain(b)
                else:
                    @pl.when(it > 0)
                    def _():
                        drain(b)
                cps = [
                    pltpu.async_copy(
                        table_hbm.at[idx_v.at[h * (SUBG // LANE) + j]],
                        bufs[b].at[pl.ds(j * LANE, LANE)],
                        sem,
                    )
                    for j in range(SUBG // LANE)
                ]
                for cp in cps:
                    cp.wait()
                pltpu.async_copy(
                    bufs[b], out_hbm.at[pl.ds(base + h * SUBG, SUBG)],
                    wsems[b])
            return carry

        lax.fori_loop(0, niter, step, 0)
        drain(0)
        drain(1)

    return pl.kernel(
        body,
        out_type=jax.ShapeDtypeStruct((b_total, 2 * H), jnp.float32),
        mesh=mesh,
        scratch_types=[
            pltpu.VMEM((KJ, LANE), jnp.int32),
            pltpu.VMEM((SUBG, 2 * H), jnp.float32),
            pltpu.VMEM((SUBG, 2 * H), jnp.float32),
            pltpu.SemaphoreType.DMA,
            pltpu.SemaphoreType.DMA,
            pltpu.SemaphoreType.DMA,
        ],
    )(table, idx_rows)


def _sc_scatter_add128(upd, idx_rows, zeros_np):
    niter = NITER_E
    mesh = plsc.VectorSubcoreMesh(core_axis_name="c", subcore_axis_name="s")
    rps = NP // 16

    def body(upd_hbm, idx_hbm, z_hbm, out_hbm, idx_v, ubuf_a, ubuf_b, shared,
             sem):
        c = lax.axis_index("c")
        s = lax.axis_index("s")
        wid = s * 2 + c
        pltpu.sync_copy(z_hbm.at[pl.ds(s * rps, rps)], shared.at[pl.ds(s * rps, rps)])
        plsc.subcore_barrier()
        bufs = (ubuf_a, ubuf_b)

        def step(it, carry):
            rowbase = (wid * niter + it) * KJ
            base = (wid * niter + it) * CH
            pltpu.sync_copy(idx_hbm.at[pl.ds(rowbase, KJ)], idx_v)
            cur = pltpu.async_copy(upd_hbm.at[pl.ds(base, LANE)], bufs[0], sem)
            for j in range(KJ):
                b = j % 2
                nxt = None
                if j + 1 < KJ:
                    nxt = pltpu.async_copy(
                        upd_hbm.at[pl.ds(base + (j + 1) * LANE, LANE)],
                        bufs[1 - b], sem)
                cur.wait()
                pltpu.sync_copy(bufs[b], shared.at[idx_v.at[j]], add=True)
                cur = nxt
            return carry

        lax.fori_loop(0, niter, step, 0)
        plsc.subcore_barrier()
        pltpu.sync_copy(
            shared.at[pl.ds(s * rps, rps)],
            out_hbm.at[pl.ds(c * NP + s * rps, rps)],
        )

    return pl.kernel(
        body,
        out_type=jax.ShapeDtypeStruct((2 * NP, 2 * H), jnp.float32),
        mesh=mesh,
        scratch_types=[
            pltpu.VMEM((KJ, LANE), jnp.int32),
            pltpu.VMEM((LANE, 2 * H), jnp.float32),
            pltpu.VMEM((LANE, 2 * H), jnp.float32),
            pltpu.VMEM_SHARED((NP, 2 * H), jnp.float32),
            pltpu.SemaphoreType.DMA,
        ],
    )(upd, idx_rows, zeros_np)


BE = 4096
BR = BE // 128


def _full(shape):
    return pl.BlockSpec(shape, lambda i: tuple(0 for _ in shape))


def _expand64(dense_ref):
    return jnp.concatenate(
        [jnp.broadcast_to(dense_ref[r:r + 1, :].T, (LANE, H)) for r in range(BR)],
        axis=0)


def _edge_featurize(comps, r1w0, r1b0, r2w0, r2b0, r1w1, r1b1, r2w1, r2b1,
                    or1w, or1b, or2w, or2b, epw, epb, awc):
    grid = (EP // BE,)

    def body(sx_ref, sy_ref, sz_ref, dx_ref, dy_ref, dz_ref,
             w10_ref, b10_ref, w20_ref, b20_ref,
             w11_ref, b11_ref, w21_ref, b21_ref,
             ow1_ref, ob1_ref, ow2_ref, ob2_ref,
             wm_ref, bm_ref, awc_ref,
             x1_ref, x2_ref, aw1_ref, aw2_ref):
        vx = sx_ref[...] - dx_ref[...]
        vy = sy_ref[...] - dy_ref[...]
        vz = sz_ref[...] - dz_ref[...]
        d2 = vx * vx + vy * vy + vz * vz + 1e-12
        inv = lax.rsqrt(d2)
        dd = d2 * inv
        x = vx * inv
        y = vy * inv
        z = vz * inv
        xc = dd * (1.0 / CUTOFF)
        xc2 = xc * xc
        xc6 = (xc2 * xc) * (xc2 * xc)
        env = 1.0 - 28.0 * xc6 + 48.0 * xc6 * xc - 21.0 * xc6 * xc2
        env = env * (dd < CUTOFF).astype(jnp.float32)
        a = dd * (_PI / CUTOFF)
        s1 = jnp.sin(a)
        c2 = 2.0 * jnp.cos(a)
        scale = (_BES_SCALE * inv) * env
        rbfs = [scale * s1]
        sprev, scur = jnp.zeros_like(s1), s1
        for _ in range(15):
            sprev, scur = scur, c2 * scur - sprev
            rbfs.append(scale * scur)
        awc = awc_ref[...]
        z2t = 3.0 * z * z - 1.0
        for b, ref in ((0, aw1_ref), (1, aw2_ref)):
            lin = (awc[0, b] + awc[1, b] * y + awc[2, b] * z + awc[3, b] * x
                   + awc[4, b] * x * y + awc[5, b] * y * z + awc[6, b] * z2t
                   + awc[7, b])
            ref[...] = _sigm(lin)
        pieces = []
        for r in range(BR):
            rt = jnp.concatenate([rb[r:r + 1, :] for rb in rbfs], axis=0)
            pieces.append(rt.T)
        R = jnp.concatenate(pieces, axis=0)
        m0 = jnp.dot(R, wm_ref[...]) + bm_ref[...]
        rf1 = jnp.dot(_silu(jnp.dot(R, w10_ref[...]) + b10_ref[...]),
                      w20_ref[...]) + b20_ref[...]
        rf2 = jnp.dot(_silu(jnp.dot(R, w11_ref[...]) + b11_ref[...]),
                      w21_ref[...]) + b21_ref[...]
        wout = jnp.dot(_silu(jnp.dot(R, ow1_ref[...]) + ob1_ref[...]),
                       ow2_ref[...]) + ob2_ref[...]
        x1_ref[...] = jnp.concatenate([m0 * rf1, wout], axis=1)
        x2_ref[...] = jnp.concatenate([rf2, m0], axis=1)

    dspec = pl.BlockSpec((BR, LANE), lambda i: (i, 0))
    return pl.pallas_call(
        body,
        grid=grid,
        in_specs=[dspec] * 6 + [
            _full((16, H)), _full((1, H)), _full((H, H)), _full((1, H)),
            _full((16, H)), _full((1, H)), _full((H, H)), _full((1, H)),
            _full((16, H)), _full((1, H)), _full((H, H)), _full((1, H)),
            _full((16, H)), _full((1, H)), _full((8, 2)),
        ],
        out_specs=[
            pl.BlockSpec((BE, 2 * H), lambda i: (i, 0)),
            pl.BlockSpec((BE, 2 * H), lambda i: (i, 0)),
            dspec, dspec,
        ],
        out_shape=[
            jax.ShapeDtypeStruct((EP, 2 * H), jnp.float32),
            jax.ShapeDtypeStruct((EP, 2 * H), jnp.float32),
            jax.ShapeDtypeStruct((EP // LANE, LANE), jnp.float32),
            jax.ShapeDtypeStruct((EP // LANE, LANE), jnp.float32),
        ],
    )(*comps, r1w0, r1b0, r2w0, r2b0, r1w1, r1b1, r2w1, r2b1,
      or1w, or1b, or2w, or2b, epw, epb, awc)


def _block1_triplet(G1, Gaw1d, X2, ow, obs):
    grid = (EP // BE,)

    l01 = jnp.concatenate([ow[0], ow[1]], axis=1)
    b01 = jnp.concatenate([obs[0:1, :], obs[1:2, :]], axis=1)

    def body(gmr_ref, gaw_ref, x2_ref,
             l01_ref, b01_ref, l2_ref, b2_ref, x5_ref):
        gid = pl.program_id(0)
        rows = gid * BE + lax.broadcasted_iota(jnp.int32, (BE, 1), 0)
        valid = (rows < N_EDGES).astype(jnp.float32)
        aw64 = _expand64(gaw_ref)
        g1 = gmr_ref[:, 0:H] * aw64 * valid
        a = _silu(jnp.dot(g1, l01_ref[...]) + b01_ref[...])
        mn = (a[:, 0:H] + a[:, H:2 * H]
              + _silu(jnp.dot(g1, l2_ref[...]) + b2_ref[...]))
        x2 = x2_ref[...]
        mr2 = (x2[:, H:2 * H] + mn) * x2[:, 0:H]
        x5_ref[...] = jnp.concatenate([g1, mr2], axis=1)

    return pl.pallas_call(
        body,
        grid=grid,
        in_specs=[
            pl.BlockSpec((BE, 2 * H), lambda i: (i, 0)),
            pl.BlockSpec((BR, LANE), lambda i: (i, 0)),
            pl.BlockSpec((BE, 2 * H), lambda i: (i, 0)),
            _full((H, 2 * H)), _full((1, 2 * H)), _full((H, H)), _full((1, H)),
        ],
        out_specs=pl.BlockSpec((BE, 2 * H), lambda i: (i, 0)),
        out_shape=jax.ShapeDtypeStruct((EP, 2 * H), jnp.float32),
    )(G1, Gaw1d, X2, l01, b01, ow[2], obs[2:3, :])


def _block2_messages(G2, Gaw2d):
    grid = (EP // BE,)

    def body(gmr_ref, gaw_ref, x7_ref):
        gid = pl.program_id(0)
        rows = gid * BE + lax.broadcasted_iota(jnp.int32, (BE, 1), 0)
        valid = (rows < N_EDGES).astype(jnp.float32)
        g2 = gmr_ref[:, H:2 * H] * _expand64(gaw_ref) * valid
        x7_ref[...] = jnp.concatenate(
            [g2, jnp.zeros((BE, H), jnp.float32)], axis=1)

    return pl.pallas_call(
        body,
        grid=grid,
        in_specs=[
            pl.BlockSpec((BE, 2 * H), lambda i: (i, 0)),
            pl.BlockSpec((BR, LANE), lambda i: (i, 0)),
        ],
        out_specs=pl.BlockSpec((BE, 2 * H), lambda i: (i, 0)),
        out_shape=jax.ShapeDtypeStruct((EP, 2 * H), jnp.float32),
    )(G2, Gaw2d)


def _out_edges(GH, X1):
    grid = (EP // BE,)

    def body(gh_ref, wo_ref, x9_ref):
        gid = pl.program_id(0)
        rows = gid * BE + lax.broadcasted_iota(jnp.int32, (BE, 1), 0)
        valid = (rows < N_EDGES).astype(jnp.float32)
        he = gh_ref[:, 0:H] * wo_ref[:, H:2 * H] * valid
        x9_ref[...] = jnp.concatenate(
            [he, jnp.zeros((BE, H), jnp.float32)], axis=1)

    return pl.pallas_call(
        body,
        grid=grid,
        in_specs=[
            pl.BlockSpec((BE, 2 * H), lambda i: (i, 0)),
            pl.BlockSpec((BE, 2 * H), lambda i: (i, 0)),
        ],
        out_specs=pl.BlockSpec((BE, 2 * H), lambda i: (i, 0)),
        out_shape=jax.ShapeDtypeStruct((EP, 2 * H), jnp.float32),
    )(GH, X1)


def _node_update(h, aggp, wh, wa, b1, w2, b2, duplicate_out):
    def body(h_ref, a0_ref, a1_ref, wh_ref, wa_ref, b1_ref, w2_ref, b2_ref,
             o_ref):
        agg = a0_ref[0:N_NODES, 0:H] + a1_ref[0:N_NODES, 0:H]
        h = h_ref[...]
        pre = _silu(jnp.dot(h, wh_ref[...]) + jnp.dot(agg, wa_ref[...])
                    + b1_ref[...])
        hn = h + jnp.dot(pre, w2_ref[...]) + b2_ref[...]
        if duplicate_out:
            o_ref[...] = jnp.concatenate([hn, hn], axis=1)
        else:
            o_ref[...] = hn

    return pl.pallas_call(
        body,
        grid=(1,),
        in_specs=[
            _full((N_NODES, H)),
            pl.BlockSpec((NP, 2 * H), lambda i: (0, 0)),
            pl.BlockSpec((NP, 2 * H), lambda i: (1, 0)),
            _full((H, H)), _full((H, H)), _full((1, H)), _full((H, H)),
            _full((1, H)),
        ],
        out_specs=_full((N_NODES, 2 * H if duplicate_out else H)),
        out_shape=jax.ShapeDtypeStruct(
            (N_NODES, 2 * H if duplicate_out else H), jnp.float32),
    )(h, aggp, aggp, wh, wa, b1, w2, b2)


def _out_mlp(aggp, d0w, d0b, d1w, d1b, d2w, d2b, fw, fb):
    def body(a0_ref, a1_ref, d0w_ref, d0b_ref, d1w_ref, d1b_ref,
             d2w_ref, d2b_ref, fw_ref, fb_ref, o_ref):
        xx = a0_ref[0:N_NODES, 0:H] + a1_ref[0:N_NODES, 0:H]
        xx = _silu(jnp.dot(xx, d0w_ref[...]) + d0b_ref[...])
        xx = _silu(jnp.dot(xx, d1w_ref[...]) + d1b_ref[...])
        xx = _silu(jnp.dot(xx, d2w_ref[...]) + d2b_ref[...])
        o_ref[...] = jnp.dot(xx, fw_ref[...]) + fb_ref[...]

    return pl.pallas_call(
        body,
        grid=(1,),
        in_specs=[
            pl.BlockSpec((NP, 2 * H), lambda i: (0, 0)),
            pl.BlockSpec((NP, 2 * H), lambda i: (1, 0)),
            _full((H, H)), _full((1, H)), _full((H, H)), _full((1, H)),
            _full((H, H)), _full((1, H)), _full((H, 1)), _full((1, 1)),
        ],
        out_specs=_full((N_NODES, 1)),
        out_shape=jax.ShapeDtypeStruct((N_NODES, 1), jnp.float32),
    )(aggp, aggp, d0w, d0b, d1w, d1b, d2w, d2b, fw, fb)


def kernel(atomic_numbers, positions, edge_index, triplets, params):
    src = edge_index[0]
    dst = edge_index[1]
    ji = triplets[:, 0]
    kj = triplets[:, 1]

    padE = EP - N_EDGES
    arE = jnp.arange(padE, dtype=jnp.int32)
    src_p = jnp.concatenate([src, arE % N_NODES])
    dst_p = jnp.concatenate([dst, arE % N_NODES])
    ji_p = jnp.concatenate([ji, arE % N_EDGES])
    kj_p = jnp.concatenate([kj, arE % N_EDGES])
    ji_r = ji_p.reshape(EP // LANE, LANE)
    kj_r = kj_p.reshape(EP // LANE, LANE)
    src_r = src_p.reshape(EP // LANE, LANE)
    dst_r = dst_p.reshape(EP // LANE, LANE)

    posx = positions[:, 0]
    posy = positions[:, 1]
    posz = positions[:, 2]
    zeros_np = jnp.zeros((NP, 2 * H), jnp.float32)
    h0 = params["atom_emb"][atomic_numbers - 1]

    b0, b1 = params["blocks"]
    ep_w = params["edge_proj"]["w"][:, 2 * H:3 * H]
    ep_b = params["edge_proj"]["b"][2 * H:3 * H][None, :]

    def lin(p):
        return p["w"], p["b"][None, :]

    r1w0, r1b0 = lin(b0["radial1"])
    r2w0, r2b0 = lin(b0["radial2"])
    r1w1, r1b1 = lin(b1["radial1"])
    r2w1, r2b1 = lin(b1["radial2"])
    awc = jnp.stack(
        [jnp.concatenate([jnp.mean(b["sph"]["w"], axis=1),
                          jnp.mean(b["sph"]["b"])[None]])
         for b in (b0, b1)], axis=1)
    ob = params["out_block"]
    or1w, or1b = lin(ob["radial1"])
    or2w, or2b = lin(ob["radial2"])
    out_ws0 = [l["w"] for l in b0["out"]]
    out_bs0 = jnp.stack([l["b"] for l in b0["out"]])

    comps1d = (list(_sc_gather_elem([posx, posy, posz], src_r, NITER_E))
               + list(_sc_gather_elem([posx, posy, posz], dst_r, NITER_E)))
    comps = [c.reshape(EP // LANE, LANE) for c in comps1d]
    X1, X2, aw1d, aw2d = _edge_featurize(
        comps, r1w0, r1b0, r2w0, r2b0, r1w1, r1b1, r2w1, r2b1,
        or1w, or1b, or2w, or2b, ep_w, ep_b, awc)
    G1 = _sc_gather128(X1, kj_r)
    Gaw1, Gaw2 = _sc_gather_elem(
        [aw1d.reshape(EP), aw2d.reshape(EP)], ji_r, NITER_E)
    Gaw1d = Gaw1.reshape(EP // LANE, LANE)
    Gaw2d = Gaw2.reshape(EP // LANE, LANE)
    X5 = _block1_triplet(G1, Gaw1d, X2, out_ws0, out_bs0)
    agg1p = _sc_scatter_add128(X5, dst_r, zeros_np)
    u0w = b0["upd1"]["w"]
    h1 = _node_update(h0, agg1p, u0w[:H], u0w[H:], b0["upd1"]["b"][None, :],
                      b0["upd2"]["w"], b0["upd2"]["b"][None, :], False)
    G2 = _sc_gather128(X5, kj_r)
    X7 = _block2_messages(G2, Gaw2d)
    agg2p = _sc_scatter_add128(X7, dst_r, zeros_np)
    u1w = b1["upd1"]["w"]
    h2x = _node_update(h1, agg2p, u1w[:H], u1w[H:], b1["upd1"]["b"][None, :],
                       b1["upd2"]["w"], b1["upd2"]["b"][None, :], True)
    GH = _sc_gather128(h2x, src_r)
    X9 = _out_edges(GH, X1)
    agg3p = _sc_scatter_add128(X9, dst_r, zeros_np)
    d0, d1, d2 = ob["dense"]
    return _out_mlp(agg3p, d0["w"], d0["b"][None, :], d1["w"], d1["b"][None, :],
                    d2["w"], d2["b"][None, :], ob["final"]["w"],
                    ob["final"]["b"][None, :])


# speedup vs baseline: 1.1041x; 1.0228x over previous
import jax
import jax.numpy as jnp
import numpy as np
from jax import lax
from jax.experimental import pallas as pl
from jax.experimental.pallas import tpu as pltpu
from jax.experimental.pallas import tpu_sc as plsc

N_NODES = 10000
N_EDGES = 320000
H = 64
CUTOFF = 5.0

NW = 32
LANE = 128
KJ = 8
CH = KJ * LANE
NITER_E = 10
EP = NW * NITER_E * CH
NP = 10240

_PI = float(np.pi)
_BES_SCALE = float(np.sqrt(2.0 / CUTOFF))


def _sigm(x):
    return 0.5 * (1.0 + jnp.tanh(0.5 * x))


def _silu(x):
    return x * _sigm(x)


def _sc_gather_elem(tables, idx_rows, niter):
    nt = len(tables)
    nrows = idx_rows.shape[0]
    b_total = nrows * LANE
    mesh = plsc.VectorSubcoreMesh(core_axis_name="c", subcore_axis_name="s")

    def body(*refs):
        t_refs = refs[:nt]
        idx_hbm = refs[nt]
        o_refs = refs[nt + 1:nt + 1 + nt]
        idx_v = refs[nt + 1 + nt]
        row_bufs = refs[nt + 2 + nt:nt + 2 + 2 * nt]
        sem, wsem = refs[nt + 2 + 2 * nt:]
        c = lax.axis_index("c")
        s = lax.axis_index("s")
        wid = s * 2 + c

        def step(it, carry):
            rowbase = (wid * niter + it) * KJ
            base = (wid * niter + it) * CH
            pltpu.sync_copy(idx_hbm.at[pl.ds(rowbase, KJ)], idx_v)
            outs = []
            for t in range(nt):
                cps = [
                    pltpu.async_copy(
                        t_refs[t].at[idx_v.at[j]],
                        row_bufs[t].at[pl.ds(j * LANE, LANE)],
                        sem,
                    )
                    for j in range(KJ)
                ]
                for cp in cps:
                    cp.wait()
                outs.append(pltpu.async_copy(
                    row_bufs[t], o_refs[t].at[pl.ds(base, CH)], wsem))
            for o in outs:
                o.wait()
            return carry

        lax.fori_loop(0, niter, step, 0)

    return pl.kernel(
        body,
        out_type=[jax.ShapeDtypeStruct((b_total,), jnp.float32)] * nt,
        mesh=mesh,
        compiler_params=pltpu.CompilerParams(use_tc_tiling_on_sc=False),
        scratch_types=[pltpu.VMEM((KJ, LANE), jnp.int32)]
        + [pltpu.VMEM((CH,), jnp.float32)] * nt
        + [pltpu.SemaphoreType.DMA, pltpu.SemaphoreType.DMA],
    )(*tables, idx_rows)


def _sc_gather128(table, idx_rows):
    niter = NITER_E
    nrows = idx_rows.shape[0]
    b_total = nrows * LANE
    mesh = plsc.VectorSubcoreMesh(core_axis_name="c", subcore_axis_name="s")

    SUBG = 256

    def body(table_hbm, idx_hbm, out_hbm, idx_v, rows_a, rows_b, sem, wsa, wsb):
        c = lax.axis_index("c")
        s = lax.axis_index("s")
        wid = s * 2 + c
        bufs = (rows_a, rows_b)
        wsems = (wsa, wsb)

        def drain(b):
            pltpu.make_async_copy(
                out_hbm.at[pl.ds(0, SUBG)], bufs[b], wsems[b]).wait()

        def step(it, carry):
            rowbase = (wid * niter + it) * KJ
            base = (wid * niter + it) * CH
            pltpu.sync_copy(idx_hbm.at[pl.ds(rowbase, KJ)], idx_v)
            for h in range(CH // SUBG):
                b = h % 2
                if h >= 2:
                    drain(b)
                else:
                    @pl.when(it > 0)
                    def _():
                        drain(b)
                cps = [
                    pltpu.async_copy(
                        table_hbm.at[idx_v.at[h * (SUBG // LANE) + j]],
                        bufs[b].at[pl.ds(j * LANE, LANE)],
                        sem,
                    )
                    for j in range(SUBG // LANE)
                ]
                for cp in cps:
                    cp.wait()
                pltpu.async_copy(
                    bufs[b], out_hbm.at[pl.ds(base + h * SUBG, SUBG)],
                    wsems[b])
            return carry

        lax.fori_loop(0, niter, step, 0)
        drain(0)
        drain(1)

    return pl.kernel(
        body,
        out_type=jax.ShapeDtypeStruct((b_total, 2 * H), jnp.float32),
        mesh=mesh,
        scratch_types=[
            pltpu.VMEM((KJ, LANE), jnp.int32),
            pltpu.VMEM((SUBG, 2 * H), jnp.float32),
            pltpu.VMEM((SUBG, 2 * H), jnp.float32),
            pltpu.SemaphoreType.DMA,
            pltpu.SemaphoreType.DMA,
            pltpu.SemaphoreType.DMA,
        ],
    )(table, idx_rows)


def _sc_scatter_add128(upd, idx_rows, zeros_np):
    niter = NITER_E
    mesh = plsc.VectorSubcoreMesh(core_axis_name="c", subcore_axis_name="s")
    rps = NP // 16

    def body(upd_hbm, idx_hbm, z_hbm, out_hbm, idx_v, ubuf_a, ubuf_b, shared,
             sem):
        c = lax.axis_index("c")
        s = lax.axis_index("s")
        wid = s * 2 + c
        pltpu.sync_copy(z_hbm.at[pl.ds(s * rps, rps)], shared.at[pl.ds(s * rps, rps)])
        plsc.subcore_barrier()
        bufs = (ubuf_a, ubuf_b)

        def step(it, carry):
            rowbase = (wid * niter + it) * KJ
            base = (wid * niter + it) * CH
            pltpu.sync_copy(idx_hbm.at[pl.ds(rowbase, KJ)], idx_v)
            cur = pltpu.async_copy(upd_hbm.at[pl.ds(base, LANE)], bufs[0], sem)
            for j in range(KJ):
                b = j % 2
                nxt = None
                if j + 1 < KJ:
                    nxt = pltpu.async_copy(
                        upd_hbm.at[pl.ds(base + (j + 1) * LANE, LANE)],
                        bufs[1 - b], sem)
                cur.wait()
                pltpu.sync_copy(bufs[b], shared.at[idx_v.at[j]], add=True)
                cur = nxt
            return carry

        lax.fori_loop(0, niter, step, 0)
        plsc.subcore_barrier()
        pltpu.sync_copy(
            shared.at[pl.ds(s * rps, rps)],
            out_hbm.at[pl.ds(c * NP + s * rps, rps)],
        )

    return pl.kernel(
        body,
        out_type=jax.ShapeDtypeStruct((2 * NP, 2 * H), jnp.float32),
        mesh=mesh,
        scratch_types=[
            pltpu.VMEM((KJ, LANE), jnp.int32),
            pltpu.VMEM((LANE, 2 * H), jnp.float32),
            pltpu.VMEM((LANE, 2 * H), jnp.float32),
            pltpu.VMEM_SHARED((NP, 2 * H), jnp.float32),
            pltpu.SemaphoreType.DMA,
        ],
    )(upd, idx_rows, zeros_np)


BE = 8192
BR = BE // 128


def _full(shape):
    return pl.BlockSpec(shape, lambda i: tuple(0 for _ in shape))


def _expand64(dense_ref):
    return jnp.concatenate(
        [jnp.broadcast_to(dense_ref[r:r + 1, :].T, (LANE, H)) for r in range(BR)],
        axis=0)


def _edge_featurize(comps, r1w0, r1b0, r2w0, r2b0, r1w1, r1b1, r2w1, r2b1,
                    or1w, or1b, or2w, or2b, epw, epb, awc):
    grid = (EP // BE,)

    def body(sx_ref, sy_ref, sz_ref, dx_ref, dy_ref, dz_ref,
             w10_ref, b10_ref, w20_ref, b20_ref,
             w11_ref, b11_ref, w21_ref, b21_ref,
             ow1_ref, ob1_ref, ow2_ref, ob2_ref,
             wm_ref, bm_ref, awc_ref,
             x1_ref, x2_ref, aw1_ref, aw2_ref):
        vx = sx_ref[...] - dx_ref[...]
        vy = sy_ref[...] - dy_ref[...]
        vz = sz_ref[...] - dz_ref[...]
        d2 = vx * vx + vy * vy + vz * vz + 1e-12
        inv = lax.rsqrt(d2)
        dd = d2 * inv
        x = vx * inv
        y = vy * inv
        z = vz * inv
        xc = dd * (1.0 / CUTOFF)
        xc2 = xc * xc
        xc6 = (xc2 * xc) * (xc2 * xc)
        env = 1.0 - 28.0 * xc6 + 48.0 * xc6 * xc - 21.0 * xc6 * xc2
        env = env * (dd < CUTOFF).astype(jnp.float32)
        a = dd * (_PI / CUTOFF)
        s1 = jnp.sin(a)
        c2 = 2.0 * jnp.cos(a)
        scale = (_BES_SCALE * inv) * env
        rbfs = [scale * s1]
        sprev, scur = jnp.zeros_like(s1), s1
        for _ in range(15):
            sprev, scur = scur, c2 * scur - sprev
            rbfs.append(scale * scur)
        awc = awc_ref[...]
        z2t = 3.0 * z * z - 1.0
        for b, ref in ((0, aw1_ref), (1, aw2_ref)):
            lin = (awc[0, b] + awc[1, b] * y + awc[2, b] * z + awc[3, b] * x
                   + awc[4, b] * x * y + awc[5, b] * y * z + awc[6, b] * z2t
                   + awc[7, b])
            ref[...] = _sigm(lin)
        pieces = []
        for r in range(BR):
            rt = jnp.concatenate([rb[r:r + 1, :] for rb in rbfs], axis=0)
            pieces.append(rt.T)
        R = jnp.concatenate(pieces, axis=0)
        m0 = jnp.dot(R, wm_ref[...]) + bm_ref[...]
        rf1 = jnp.dot(_silu(jnp.dot(R, w10_ref[...]) + b10_ref[...]),
                      w20_ref[...]) + b20_ref[...]
        rf2 = jnp.dot(_silu(jnp.dot(R, w11_ref[...]) + b11_ref[...]),
                      w21_ref[...]) + b21_ref[...]
        wout = jnp.dot(_silu(jnp.dot(R, ow1_ref[...]) + ob1_ref[...]),
                       ow2_ref[...]) + ob2_ref[...]
        x1_ref[...] = jnp.concatenate([m0 * rf1, wout], axis=1)
        x2_ref[...] = jnp.concatenate([rf2, m0], axis=1)

    dspec = pl.BlockSpec((BR, LANE), lambda i: (i, 0))
    return pl.pallas_call(
        body,
        grid=grid,
        in_specs=[dspec] * 6 + [
            _full((16, H)), _full((1, H)), _full((H, H)), _full((1, H)),
            _full((16, H)), _full((1, H)), _full((H, H)), _full((1, H)),
            _full((16, H)), _full((1, H)), _full((H, H)), _full((1, H)),
            _full((16, H)), _full((1, H)), _full((8, 2)),
        ],
        out_specs=[
            pl.BlockSpec((BE, 2 * H), lambda i: (i, 0)),
            pl.BlockSpec((BE, 2 * H), lambda i: (i, 0)),
            dspec, dspec,
        ],
        out_shape=[
            jax.ShapeDtypeStruct((EP, 2 * H), jnp.float32),
            jax.ShapeDtypeStruct((EP, 2 * H), jnp.float32),
            jax.ShapeDtypeStruct((EP // LANE, LANE), jnp.float32),
            jax.ShapeDtypeStruct((EP // LANE, LANE), jnp.float32),
        ],
    )(*comps, r1w0, r1b0, r2w0, r2b0, r1w1, r1b1, r2w1, r2b1,
      or1w, or1b, or2w, or2b, epw, epb, awc)


def _block1_triplet(G1, Gaw1d, X2, ow, obs):
    grid = (EP // BE,)

    l01 = jnp.concatenate([ow[0], ow[1]], axis=1)
    b01 = jnp.concatenate([obs[0:1, :], obs[1:2, :]], axis=1)

    def body(gmr_ref, gaw_ref, x2_ref,
             l01_ref, b01_ref, l2_ref, b2_ref, x5_ref):
        gid = pl.program_id(0)
        rows = gid * BE + lax.broadcasted_iota(jnp.int32, (BE, 1), 0)
        valid = (rows < N_EDGES).astype(jnp.float32)
        aw64 = _expand64(gaw_ref)
        g1 = gmr_ref[:, 0:H] * aw64 * valid
        a = _silu(jnp.dot(g1, l01_ref[...]) + b01_ref[...])
        mn = (a[:, 0:H] + a[:, H:2 * H]
              + _silu(jnp.dot(g1, l2_ref[...]) + b2_ref[...]))
        x2 = x2_ref[...]
        mr2 = (x2[:, H:2 * H] + mn) * x2[:, 0:H]
        x5_ref[...] = jnp.concatenate([g1, mr2], axis=1)

    return pl.pallas_call(
        body,
        grid=grid,
        in_specs=[
            pl.BlockSpec((BE, 2 * H), lambda i: (i, 0)),
            pl.BlockSpec((BR, LANE), lambda i: (i, 0)),
            pl.BlockSpec((BE, 2 * H), lambda i: (i, 0)),
            _full((H, 2 * H)), _full((1, 2 * H)), _full((H, H)), _full((1, H)),
        ],
        out_specs=pl.BlockSpec((BE, 2 * H), lambda i: (i, 0)),
        out_shape=jax.ShapeDtypeStruct((EP, 2 * H), jnp.float32),
    )(G1, Gaw1d, X2, l01, b01, ow[2], obs[2:3, :])


def _block2_messages(G2, Gaw2d):
    grid = (EP // BE,)

    def body(gmr_ref, gaw_ref, x7_ref):
        gid = pl.program_id(0)
        rows = gid * BE + lax.broadcasted_iota(jnp.int32, (BE, 1), 0)
        valid = (rows < N_EDGES).astype(jnp.float32)
        g2 = gmr_ref[:, H:2 * H] * _expand64(gaw_ref) * valid
        x7_ref[...] = jnp.concatenate(
            [g2, jnp.zeros((BE, H), jnp.float32)], axis=1)

    return pl.pallas_call(
        body,
        grid=grid,
        in_specs=[
            pl.BlockSpec((BE, 2 * H), lambda i: (i, 0)),
            pl.BlockSpec((BR, LANE), lambda i: (i, 0)),
        ],
        out_specs=pl.BlockSpec((BE, 2 * H), lambda i: (i, 0)),
        out_shape=jax.ShapeDtypeStruct((EP, 2 * H), jnp.float32),
    )(G2, Gaw2d)


def _out_edges(GH, X1):
    grid = (EP // BE,)

    def body(gh_ref, wo_ref, x9_ref):
        gid = pl.program_id(0)
        rows = gid * BE + lax.broadcasted_iota(jnp.int32, (BE, 1), 0)
        valid = (rows < N_EDGES).astype(jnp.float32)
        he = gh_ref[:, 0:H] * wo_ref[:, H:2 * H] * valid
        x9_ref[...] = jnp.concatenate(
            [he, jnp.zeros((BE, H), jnp.float32)], axis=1)

    return pl.pallas_call(
        body,
        grid=grid,
        in_specs=[
            pl.BlockSpec((BE, 2 * H), lambda i: (i, 0)),
            pl.BlockSpec((BE, 2 * H), lambda i: (i, 0)),
        ],
        out_specs=pl.BlockSpec((BE, 2 * H), lambda i: (i, 0)),
        out_shape=jax.ShapeDtypeStruct((EP, 2 * H), jnp.float32),
    )(GH, X1)


def _node_update(h, aggp, wh, wa, b1, w2, b2, duplicate_out):
    def body(h_ref, a0_ref, a1_ref, wh_ref, wa_ref, b1_ref, w2_ref, b2_ref,
             o_ref):
        agg = a0_ref[0:N_NODES, 0:H] + a1_ref[0:N_NODES, 0:H]
        h = h_ref[...]
        pre = _silu(jnp.dot(h, wh_ref[...]) + jnp.dot(agg, wa_ref[...])
                    + b1_ref[...])
        hn = h + jnp.dot(pre, w2_ref[...]) + b2_ref[...]
        if duplicate_out:
            o_ref[...] = jnp.concatenate([hn, hn], axis=1)
        else:
            o_ref[...] = hn

    return pl.pallas_call(
        body,
        grid=(1,),
        in_specs=[
            _full((N_NODES, H)),
            pl.BlockSpec((NP, 2 * H), lambda i: (0, 0)),
            pl.BlockSpec((NP, 2 * H), lambda i: (1, 0)),
            _full((H, H)), _full((H, H)), _full((1, H)), _full((H, H)),
            _full((1, H)),
        ],
        out_specs=_full((N_NODES, 2 * H if duplicate_out else H)),
        out_shape=jax.ShapeDtypeStruct(
            (N_NODES, 2 * H if duplicate_out else H), jnp.float32),
    )(h, aggp, aggp, wh, wa, b1, w2, b2)


def _out_mlp(aggp, d0w, d0b, d1w, d1b, d2w, d2b, fw, fb):
    def body(a0_ref, a1_ref, d0w_ref, d0b_ref, d1w_ref, d1b_ref,
             d2w_ref, d2b_ref, fw_ref, fb_ref, o_ref):
        xx = a0_ref[0:N_NODES, 0:H] + a1_ref[0:N_NODES, 0:H]
        xx = _silu(jnp.dot(xx, d0w_ref[...]) + d0b_ref[...])
        xx = _silu(jnp.dot(xx, d1w_ref[...]) + d1b_ref[...])
        xx = _silu(jnp.dot(xx, d2w_ref[...]) + d2b_ref[...])
        o_ref[...] = jnp.dot(xx, fw_ref[...]) + fb_ref[...]

    return pl.pallas_call(
        body,
        grid=(1,),
        in_specs=[
            pl.BlockSpec((NP, 2 * H), lambda i: (0, 0)),
            pl.BlockSpec((NP, 2 * H), lambda i: (1, 0)),
            _full((H, H)), _full((1, H)), _full((H, H)), _full((1, H)),
            _full((H, H)), _full((1, H)), _full((H, 1)), _full((1, 1)),
        ],
        out_specs=_full((N_NODES, 1)),
        out_shape=jax.ShapeDtypeStruct((N_NODES, 1), jnp.float32),
    )(aggp, aggp, d0w, d0b, d1w, d1b, d2w, d2b, fw, fb)


def kernel(atomic_numbers, positions, edge_index, triplets, params):
    src = edge_index[0]
    dst = edge_index[1]
    ji = triplets[:, 0]
    kj = triplets[:, 1]

    padE = EP - N_EDGES
    arE = jnp.arange(padE, dtype=jnp.int32)
    src_p = jnp.concatenate([src, arE % N_NODES])
    dst_p = jnp.concatenate([dst, arE % N_NODES])
    ji_p = jnp.concatenate([ji, arE % N_EDGES])
    kj_p = jnp.concatenate([kj, arE % N_EDGES])
    ji_r = ji_p.reshape(EP // LANE, LANE)
    kj_r = kj_p.reshape(EP // LANE, LANE)
    src_r = src_p.reshape(EP // LANE, LANE)
    dst_r = dst_p.reshape(EP // LANE, LANE)

    posx = positions[:, 0]
    posy = positions[:, 1]
    posz = positions[:, 2]
    zeros_np = jnp.zeros((NP, 2 * H), jnp.float32)
    h0 = params["atom_emb"][atomic_numbers - 1]

    b0, b1 = params["blocks"]
    ep_w = params["edge_proj"]["w"][:, 2 * H:3 * H]
    ep_b = params["edge_proj"]["b"][2 * H:3 * H][None, :]

    def lin(p):
        return p["w"], p["b"][None, :]

    r1w0, r1b0 = lin(b0["radial1"])
    r2w0, r2b0 = lin(b0["radial2"])
    r1w1, r1b1 = lin(b1["radial1"])
    r2w1, r2b1 = lin(b1["radial2"])
    awc = jnp.stack(
        [jnp.concatenate([jnp.mean(b["sph"]["w"], axis=1),
                          jnp.mean(b["sph"]["b"])[None]])
         for b in (b0, b1)], axis=1)
    ob = params["out_block"]
    or1w, or1b = lin(ob["radial1"])
    or2w, or2b = lin(ob["radial2"])
    out_ws0 = [l["w"] for l in b0["out"]]
    out_bs0 = jnp.stack([l["b"] for l in b0["out"]])

    comps1d = (list(_sc_gather_elem([posx, posy, posz], src_r, NITER_E))
               + list(_sc_gather_elem([posx, posy, posz], dst_r, NITER_E)))
    comps = [c.reshape(EP // LANE, LANE) for c in comps1d]
    X1, X2, aw1d, aw2d = _edge_featurize(
        comps, r1w0, r1b0, r2w0, r2b0, r1w1, r1b1, r2w1, r2b1,
        or1w, or1b, or2w, or2b, ep_w, ep_b, awc)
    G1 = _sc_gather128(X1, kj_r)
    Gaw1, Gaw2 = _sc_gather_elem(
        [aw1d.reshape(EP), aw2d.reshape(EP)], ji_r, NITER_E)
    Gaw1d = Gaw1.reshape(EP // LANE, LANE)
    Gaw2d = Gaw2.reshape(EP // LANE, LANE)
    X5 = _block1_triplet(G1, Gaw1d, X2, out_ws0, out_bs0)
    agg1p = _sc_scatter_add128(X5, dst_r, zeros_np)
    u0w = b0["upd1"]["w"]
    h1 = _node_update(h0, agg1p, u0w[:H], u0w[H:], b0["upd1"]["b"][None, :],
                      b0["upd2"]["w"], b0["upd2"]["b"][None, :], False)
    G2 = _sc_gather128(X5, kj_r)
    X7 = _block2_messages(G2, Gaw2d)
    agg2p = _sc_scatter_add128(X7, dst_r, zeros_np)
    u1w = b1["upd1"]["w"]
    h2x = _node_update(h1, agg2p, u1w[:H], u1w[H:], b1["upd1"]["b"][None, :],
                       b1["upd2"]["w"], b1["upd2"]["b"][None, :], True)
    GH = _sc_gather128(h2x, src_r)
    X9 = _out_edges(GH, X1)
    agg3p = _sc_scatter_add128(X9, dst_r, zeros_np)
    d0, d1, d2 = ob["dense"]
    return _out_mlp(agg3p, d0["w"], d0["b"][None, :], d1["w"], d1["b"][None, :],
                    d2["w"], d2["b"][None, :], ob["final"]["w"],
                    ob["final"]["b"][None, :])
